# baseline (speedup 1.0000x reference)
import jax
import jax.numpy as jnp
from jax.experimental import pallas as pl
from jax.experimental.pallas import tpu as pltpu

B, N, D = 64, 128, 128
BB = 16
F32 = jnp.float32


def _gnn_kernel(sizes_ref,
                f1_ref, a1_ref, f2_ref, a2_ref, w_ref,
                out_ref):
    w1_ref = w_ref.at[0]
    w2_ref = w_ref.at[1]
    wa_ref = w_ref.at[2]
    wc1_ref = w_ref.at[3]
    wc2_ref = w_ref.at[4]
    pid = pl.program_id(0)
    lane_iota = jax.lax.broadcasted_iota(jnp.int32, (1, N), 1)
    dot = lambda a, b_: jnp.dot(a, b_, preferred_element_type=F32)

    sizes1 = [sizes_ref[pid * BB + i] for i in range(BB)]
    sizes2 = [sizes_ref[B + pid * BB + i] for i in range(BB)]
    cms = ([(lane_iota < s).astype(F32) for s in sizes1]
           + [(lane_iota < s).astype(F32) for s in sizes2])

    As = ([a1_ref[i] * cms[i] for i in range(BB)]
          + [a2_ref[i] * cms[BB + i] for i in range(BB)])

    h1 = dot(f1_ref[...].reshape(BB * N, D), w1_ref[...])
    h2 = dot(f2_ref[...].reshape(BB * N, D), w1_ref[...])
    t = ([jnp.maximum(dot(As[k], h1[k * N:(k + 1) * N]), 0.0)
          for k in range(BB)]
         + [jnp.maximum(dot(As[BB + k], h2[k * N:(k + 1) * N]), 0.0)
            for k in range(BB)])

    u = [dot(t[k], w2_ref[...]) for k in range(2 * BB)]
    v = [jnp.maximum(dot(As[k], u[k]), 0.0) for k in range(2 * BB)]

    S1 = jnp.concatenate([dot(cms[k], v[k]) for k in range(BB)], axis=0)
    S2 = jnp.concatenate([dot(cms[BB + k], v[BB + k]) for k in range(BB)],
                         axis=0)

    inv1 = jnp.concatenate(
        [(1.0 / jnp.maximum(s, 1).astype(F32)).reshape(1, 1) for s in sizes1],
        axis=0)
    inv2 = jnp.concatenate(
        [(1.0 / jnp.maximum(s, 1).astype(F32)).reshape(1, 1) for s in sizes2],
        axis=0)

    emb1 = dot(S1, wa_ref[...]) * inv1
    emb2 = dot(S2, wa_ref[...]) * inv2

    r = dot(emb1, wc1_ref[...]) + dot(emb2, wc2_ref[...])
    out_ref[...] = r.reshape(BB, 1, D)


def kernel(feats_1, adjs_1, feats_2, adjs_2, sizes_1, sizes_2,
           W1, b1, W2, b2, Wa, ba, Wc, bc):
    sizes = jnp.concatenate([sizes_1.astype(jnp.int32),
                             sizes_2.astype(jnp.int32)])

    C = Wc.shape[1]
    wc1 = jnp.pad(Wc[:D], ((0, 0), (0, D - C)))
    wc2 = jnp.pad(Wc[D:], ((0, 0), (0, D - C)))
    wstk = jnp.stack([W1, W2, Wa, wc1, wc2])
    batch_spec = pl.BlockSpec((BB, N, D), lambda b: (b, 0, 0))
    w_spec = pl.BlockSpec((5, D, D), lambda b: (0, 0, 0))
    smem_spec = pl.BlockSpec(memory_space=pltpu.SMEM)

    out3 = pl.pallas_call(
        _gnn_kernel,
        grid=(B // BB,),
        in_specs=[smem_spec,
                  batch_spec, batch_spec, batch_spec, batch_spec,
                  w_spec],
        out_specs=pl.BlockSpec((BB, 1, D), lambda b: (b, 0, 0)),
        out_shape=jax.ShapeDtypeStruct((B, 1, D), F32),
        compiler_params=pltpu.CompilerParams(
            dimension_semantics=("parallel",)),
    )(sizes,
      feats_1, adjs_1, feats_2, adjs_2, wstk)

    return out3.reshape(B, D)[:, :C]


# speedup vs baseline: 1.2230x; 1.2230x over previous
import jax
import jax.numpy as jnp
from jax.experimental import pallas as pl
from jax.experimental.pallas import tpu as pltpu

B, N, D = 64, 128, 128
C_OUT = 2
BB = 16
F32 = jnp.float32


def _gnn_kernel(sizes1_ref, sizes2_ref,
                f1_ref, a1_ref, f2_ref, a2_ref,
                w1_ref, w2_ref, wa_ref, wc_ref,
                out_ref):
    pid = pl.program_id(0)
    lane_iota = jax.lax.broadcasted_iota(jnp.int32, (1, N), 1)
    dot = lambda a, b_: jnp.dot(a, b_, preferred_element_type=F32)

    sizes1 = [sizes1_ref[pid * BB + i] for i in range(BB)]
    sizes2 = [sizes2_ref[pid * BB + i] for i in range(BB)]
    cms = ([(lane_iota < s).astype(F32) for s in sizes1]
           + [(lane_iota < s).astype(F32) for s in sizes2])

    As = ([a1_ref[i] * cms[i] for i in range(BB)]
          + [a2_ref[i] * cms[BB + i] for i in range(BB)])

    h1 = dot(f1_ref[...].reshape(BB * N, D), w1_ref[...])
    h2 = dot(f2_ref[...].reshape(BB * N, D), w1_ref[...])
    t = ([jnp.maximum(dot(As[k], h1[k * N:(k + 1) * N]), 0.0)
          for k in range(BB)]
         + [jnp.maximum(dot(As[BB + k], h2[k * N:(k + 1) * N]), 0.0)
            for k in range(BB)])

    u = [dot(t[k], w2_ref[...]) for k in range(2 * BB)]
    v = [jnp.maximum(dot(As[k], u[k]), 0.0) for k in range(2 * BB)]

    S1 = jnp.concatenate([dot(cms[k], v[k]) for k in range(BB)], axis=0)
    S2 = jnp.concatenate([dot(cms[BB + k], v[BB + k]) for k in range(BB)],
                         axis=0)

    inv1 = jnp.concatenate(
        [(1.0 / jnp.maximum(s, 1).astype(F32)).reshape(1, 1) for s in sizes1],
        axis=0)
    inv2 = jnp.concatenate(
        [(1.0 / jnp.maximum(s, 1).astype(F32)).reshape(1, 1) for s in sizes2],
        axis=0)

    emb1 = dot(S1, wa_ref[...]) * inv1
    emb2 = dot(S2, wa_ref[...]) * inv2

    r = dot(jnp.concatenate([emb1, emb2], axis=1), wc_ref[...])
    out_ref[...] = r.reshape(BB, 1, C_OUT)


def kernel(feats_1, adjs_1, feats_2, adjs_2, sizes_1, sizes_2,
           W1, b1, W2, b2, Wa, ba, Wc, bc):
    sizes_1 = sizes_1.astype(jnp.int32)
    sizes_2 = sizes_2.astype(jnp.int32)

    C = Wc.shape[1]
    wc1 = jnp.pad(Wc[:D], ((0, 0), (0, D - C)))
    wc2 = jnp.pad(Wc[D:], ((0, 0), (0, D - C)))
    batch_spec = pl.BlockSpec((BB, N, D), lambda b: (b, 0, 0))
    w_spec = pl.BlockSpec((D, D), lambda b: (0, 0))
    wc_spec = pl.BlockSpec((2 * D, C_OUT), lambda b: (0, 0))
    smem_spec = pl.BlockSpec(memory_space=pltpu.SMEM)

    out3 = pl.pallas_call(
        _gnn_kernel,
        grid=(B // BB,),
        in_specs=[smem_spec, smem_spec,
                  batch_spec, batch_spec, batch_spec, batch_spec,
                  w_spec, w_spec, w_spec, wc_spec],
        out_specs=pl.BlockSpec((BB, 1, C_OUT), lambda b: (b, 0, 0)),
        out_shape=jax.ShapeDtypeStruct((B, 1, C_OUT), F32),
        compiler_params=pltpu.CompilerParams(
            dimension_semantics=("parallel",)),
    )(sizes_1, sizes_2,
      feats_1, adjs_1, feats_2, adjs_2,
      W1, W2, Wa, Wc)

    return out3.reshape(B, C_OUT)
